# Initial kernel scaffold; baseline (speedup 1.0000x reference)
import functools

import jax
import jax.numpy as jnp
from jax import lax
from jax.experimental import pallas as pl
from jax.experimental.pallas import tpu as pltpu
from jax.experimental.pallas import tpu_sc as plsc

VOCAB = 1000000
EMB = 32
IMG_F = 512
OUT = 1000
B = 4096
L = 200

NC = 2
NS = 16
NW = NC * NS
ROWS_PER_W = B // NW
R = 4
NCHUNK = ROWS_PER_W // R
IDX_W = 40
GPC = (R * L) // IDX_W


def _cbow_body(words2, table, h_out, idx0, idx1, rows0, rows1, hbuf, t0buf,
               sem0, sem1):
    wid = lax.axis_index("s") * NC + lax.axis_index("c")
    base_row = wid * ROWS_PER_W
    idx_bufs = (idx0, idx1)
    rows_bufs = (rows0, rows1)
    sems = (sem0, sem1)

    pltpu.sync_copy(table.at[pl.ds(0, 1)], t0buf)
    t0a = t0buf[0, 0:16]
    t0b = t0buf[0, 16:32]

    def fire(c, buf):
        start = (base_row + c * R) * (L // IDX_W)
        pltpu.sync_copy(words2.at[pl.ds(start, GPC)], idx_bufs[buf])
        for j in range(GPC):
            pltpu.async_copy(
                table.at[idx_bufs[buf].at[j]],
                rows_bufs[buf].at[pl.ds(j * IDX_W, IDX_W)],
                sems[buf],
            )

    def drain(buf):
        pltpu.make_async_copy(
            table.at[pl.ds(0, R * L)], rows_bufs[buf], sems[buf]
        ).wait()

    lanes = lax.iota(jnp.int32, 16)

    def process(c, buf):
        idx_v = idx_bufs[buf]
        rows_v = rows_bufs[buf]
        for r in range(R):
            cz = jnp.zeros((16,), jnp.int32)
            one = jnp.ones((16,), jnp.int32)
            zero = jnp.zeros((16,), jnp.int32)
            for jj in range(L // IDX_W):
                j = r * (L // IDX_W) + jj
                x1 = idx_v[j, 0:16]
                x2 = idx_v[j, 16:32]
                x3 = idx_v[j, 24:40]
                cz = cz + jnp.where(x1 == 0, one, zero)
                cz = cz + jnp.where(x2 == 0, one, zero)
                cz = cz + jnp.where((x3 == 0) & (lanes >= 8), one, zero)
            cnt = jnp.sum(cz).astype(jnp.float32)

            zf = jnp.zeros((16,), jnp.float32)

            def body(i, accs):
                a0, a1, a2, a3 = accs
                row = r * L + 2 * i
                a0 = a0 + rows_v[row, 0:16]
                a1 = a1 + rows_v[row, 16:32]
                a2 = a2 + rows_v[row + 1, 0:16]
                a3 = a3 + rows_v[row + 1, 16:32]
                return (a0, a1, a2, a3)

            a0, a1, a2, a3 = lax.fori_loop(0, L // 2, body, (zf, zf, zf, zf))
            rloc = c * R + r
            hbuf[rloc, 0:16] = a0 + a2 - cnt * t0a
            hbuf[rloc, 16:32] = a1 + a3 - cnt * t0b

    fire(0, 0)

    def outer(cc, carry):
        c0 = 2 * cc
        fire(c0 + 1, 1)
        drain(0)
        process(c0, 0)
        fire(c0 + 2, 0)
        drain(1)
        process(c0 + 1, 1)
        return carry

    lax.fori_loop(0, NCHUNK // 2 - 1, outer, 0)
    c0 = NCHUNK - 2
    fire(c0 + 1, 1)
    drain(0)
    process(c0, 0)
    drain(1)
    process(c0 + 1, 1)

    pltpu.sync_copy(hbuf, h_out.at[pl.ds(base_row, ROWS_PER_W)])


def _cbow_sum(words, table):
    words2 = words.reshape(B * (L // IDX_W), IDX_W)
    mesh = plsc.VectorSubcoreMesh(core_axis_name="c", subcore_axis_name="s")
    kern = pl.kernel(
        _cbow_body,
        out_type=jax.ShapeDtypeStruct((B, EMB), jnp.float32),
        mesh=mesh,
        scratch_types=[
            pltpu.VMEM((GPC, IDX_W), jnp.int32),
            pltpu.VMEM((GPC, IDX_W), jnp.int32),
            pltpu.VMEM((R * L, EMB), jnp.float32),
            pltpu.VMEM((R * L, EMB), jnp.float32),
            pltpu.VMEM((ROWS_PER_W, EMB), jnp.float32),
            pltpu.VMEM((1, EMB), jnp.float32),
            pltpu.SemaphoreType.DMA,
            pltpu.SemaphoreType.DMA,
        ],
    )
    return kern(words2, table)


def _mlp_body(image_ref, h_ref, w1_ref, b1_ref, out_ref):
    x = image_ref[...]
    h = h_ref[...]
    wa = w1_ref[:, 0:IMG_F]
    wb = w1_ref[:, IMG_F:IMG_F + EMB]
    acc = lax.dot_general(x, wa, (((1,), (1,)), ((), ())),
                          preferred_element_type=jnp.float32)
    acc = acc + lax.dot_general(h, wb, (((1,), (1,)), ((), ())),
                                preferred_element_type=jnp.float32)
    out_ref[...] = acc + b1_ref[...][None, :]


def _mlp(image, h, W1, b1):
    BLK = 1024
    grid = (B // BLK,)
    return pl.pallas_call(
        _mlp_body,
        grid=grid,
        in_specs=[
            pl.BlockSpec((BLK, IMG_F), lambda i: (i, 0)),
            pl.BlockSpec((BLK, EMB), lambda i: (i, 0)),
            pl.BlockSpec((OUT, IMG_F + EMB), lambda i: (0, 0)),
            pl.BlockSpec((OUT,), lambda i: (0,)),
        ],
        out_specs=pl.BlockSpec((BLK, OUT), lambda i: (i, 0)),
        out_shape=jax.ShapeDtypeStruct((B, OUT), jnp.float32),
    )(image, h, W1, b1)


@jax.jit
def kernel(words, image, table, W1, b1):
    h = _cbow_sum(words.astype(jnp.int32), table)
    return _mlp(image, h, W1, b1)


# speedup vs baseline: 2.4007x; 2.4007x over previous
import jax
import jax.numpy as jnp
from jax import lax
from jax.experimental import pallas as pl
from jax.experimental.pallas import tpu as pltpu
from jax.experimental.pallas import tpu_sc as plsc

VOCAB = 1000000
EMB = 32
IMG_F = 512
OUT = 1000
B = 4096
L = 200

NC = 2
NS = 16
NW = NC * NS
ROWS_PER_W = B // NW
R = 8
NCHUNK = ROWS_PER_W // R
CI = R * L
GW = 128
NG = CI // GW


def _cbow_body(words_f, table, h_out, idx0, idx1, rows0, rows1, hbuf,
               sem0, sem1):
    wid = lax.axis_index("s") * NC + lax.axis_index("c")
    base_idx = wid * ROWS_PER_W * L
    idx_bufs = (idx0, idx1)
    rows_bufs = (rows0, rows1)
    sems = (sem0, sem1)

    def fire(c, buf):
        start = base_idx + c * CI
        pltpu.sync_copy(words_f.at[pl.ds(start, CI)],
                        idx_bufs[buf].at[pl.ds(0, CI)])
        for j in range(NG):
            pltpu.async_copy(
                table.at[idx_bufs[buf].at[pl.ds(j * GW, GW)]],
                rows_bufs[buf].at[pl.ds(j * GW, GW)],
                sems[buf],
            )
        pltpu.async_copy(
            table.at[idx_bufs[buf].at[pl.ds(NG * GW, CI - NG * GW)]],
            rows_bufs[buf].at[pl.ds(NG * GW, CI - NG * GW)],
            sems[buf],
        )

    def drain(buf):
        pltpu.make_async_copy(
            table.at[pl.ds(0, CI)], rows_bufs[buf], sems[buf]
        ).wait()

    def process(c, buf):
        rows_v = rows_bufs[buf]
        for r in range(R):
            zf = jnp.zeros((16,), jnp.float32)

            def body(i, accs):
                a0, a1, a2, a3 = accs
                row = r * L + 2 * i
                a0 = a0 + rows_v[row, 0:16]
                a1 = a1 + rows_v[row, 16:32]
                a2 = a2 + rows_v[row + 1, 0:16]
                a3 = a3 + rows_v[row + 1, 16:32]
                return (a0, a1, a2, a3)

            a0, a1, a2, a3 = lax.fori_loop(0, L // 2, body, (zf, zf, zf, zf))
            rloc = c * R + r
            hbuf[rloc, 0:16] = a0 + a2
            hbuf[rloc, 16:32] = a1 + a3

    fire(0, 0)

    def outer(cc, carry):
        c0 = 2 * cc
        fire(c0 + 1, 1)
        drain(0)
        process(c0, 0)
        fire(c0 + 2, 0)
        drain(1)
        process(c0 + 1, 1)
        return carry

    lax.fori_loop(0, NCHUNK // 2 - 1, outer, 0)
    c0 = NCHUNK - 2
    fire(c0 + 1, 1)
    drain(0)
    process(c0, 0)
    drain(1)
    process(c0 + 1, 1)

    pltpu.sync_copy(hbuf, h_out.at[pl.ds(wid * ROWS_PER_W, ROWS_PER_W)])


def _cbow_sum(words, table):
    words_f = words.reshape(B * L)
    mesh = plsc.VectorSubcoreMesh(core_axis_name="c", subcore_axis_name="s")
    kern = pl.kernel(
        _cbow_body,
        out_type=jax.ShapeDtypeStruct((B, EMB), jnp.float32),
        mesh=mesh,
        scratch_types=[
            pltpu.VMEM((CI + 16,), jnp.int32),
            pltpu.VMEM((CI + 16,), jnp.int32),
            pltpu.VMEM((CI, EMB), jnp.float32),
            pltpu.VMEM((CI, EMB), jnp.float32),
            pltpu.VMEM((ROWS_PER_W, EMB), jnp.float32),
            pltpu.SemaphoreType.DMA,
            pltpu.SemaphoreType.DMA,
        ],
        compiler_params=pltpu.CompilerParams(use_tc_tiling_on_sc=False),
    )
    return kern(words_f, table)


def _mlp_body(image_ref, h_ref, words_ref, t0_ref, w1_ref, b1_ref, out_ref):
    x = image_ref[...]
    cnt = jnp.sum((words_ref[...] == 0).astype(jnp.float32), axis=1,
                  keepdims=True)
    h = h_ref[...] - cnt * t0_ref[0:1, :]
    wa = w1_ref[:, 0:IMG_F]
    wb = w1_ref[:, IMG_F:IMG_F + EMB]
    acc = lax.dot_general(x, wa, (((1,), (1,)), ((), ())),
                          preferred_element_type=jnp.float32)
    acc = acc + lax.dot_general(h, wb, (((1,), (1,)), ((), ())),
                                preferred_element_type=jnp.float32)
    out_ref[...] = acc + b1_ref[...][None, :]


def _mlp(image, h, words, t0, W1, b1):
    BLK = 1024
    grid = (B // BLK,)
    return pl.pallas_call(
        _mlp_body,
        grid=grid,
        in_specs=[
            pl.BlockSpec((BLK, IMG_F), lambda i: (i, 0)),
            pl.BlockSpec((BLK, EMB), lambda i: (i, 0)),
            pl.BlockSpec((BLK, L), lambda i: (i, 0)),
            pl.BlockSpec((8, EMB), lambda i: (0, 0)),
            pl.BlockSpec((OUT, IMG_F + EMB), lambda i: (0, 0)),
            pl.BlockSpec((OUT,), lambda i: (0,)),
        ],
        out_specs=pl.BlockSpec((BLK, OUT), lambda i: (i, 0)),
        out_shape=jax.ShapeDtypeStruct((B, OUT), jnp.float32),
    )(image, h, words, t0, W1, b1)


@jax.jit
def kernel(words, image, table, W1, b1):
    words = words.astype(jnp.int32)
    h = _cbow_sum(words, table)
    return _mlp(image, h, words, table[0:8], W1, b1)


# speedup vs baseline: 2.4025x; 1.0007x over previous
import jax
import jax.numpy as jnp
from jax import lax
from jax.experimental import pallas as pl
from jax.experimental.pallas import tpu as pltpu
from jax.experimental.pallas import tpu_sc as plsc

VOCAB = 1000000
EMB = 32
IMG_F = 512
OUT = 1000
B = 4096
L = 200

NC = 2
NS = 16
NW = NC * NS
ROWS_PER_W = B // NW
R = 8
NCHUNK = ROWS_PER_W // R
CI = R * L
GW = 128
NG = CI // GW


def _cbow_body(words2, table, h_out, idx0, idx1, rows0, rows1, hbuf,
               sem0, sem1):
    wid = lax.axis_index("s") * NC + lax.axis_index("c")
    base_row = wid * ROWS_PER_W
    idx_bufs = (idx0, idx1)
    rows_bufs = (rows0, rows1)
    sems = (sem0, sem1)

    def fire(c, buf):
        row0 = base_row + c * R
        pltpu.sync_copy(words2.at[pl.ds(row0, R), :], idx_bufs[buf])
        for r in range(R):
            pltpu.async_copy(
                table.at[idx_bufs[buf].at[r, pl.ds(0, GW)]],
                rows_bufs[buf].at[pl.ds(r * L, GW)],
                sems[buf],
            )
            pltpu.async_copy(
                table.at[idx_bufs[buf].at[r, pl.ds(GW, L - GW)]],
                rows_bufs[buf].at[pl.ds(r * L + GW, L - GW)],
                sems[buf],
            )

    def drain(buf):
        pltpu.make_async_copy(
            table.at[pl.ds(0, CI)], rows_bufs[buf], sems[buf]
        ).wait()

    def process(c, buf):
        rows_v = rows_bufs[buf]
        for r in range(R):
            zf = jnp.zeros((16,), jnp.float32)

            def body(i, accs):
                a0, a1, a2, a3 = accs
                row = r * L + 2 * i
                a0 = a0 + rows_v[row, 0:16]
                a1 = a1 + rows_v[row, 16:32]
                a2 = a2 + rows_v[row + 1, 0:16]
                a3 = a3 + rows_v[row + 1, 16:32]
                return (a0, a1, a2, a3)

            a0, a1, a2, a3 = lax.fori_loop(0, L // 2, body, (zf, zf, zf, zf))
            rloc = c * R + r
            hbuf[rloc, 0:16] = a0 + a2
            hbuf[rloc, 16:32] = a1 + a3

    fire(0, 0)

    def outer(cc, carry):
        c0 = 2 * cc
        fire(c0 + 1, 1)
        drain(0)
        process(c0, 0)
        fire(c0 + 2, 0)
        drain(1)
        process(c0 + 1, 1)
        return carry

    lax.fori_loop(0, NCHUNK // 2 - 1, outer, 0)
    c0 = NCHUNK - 2
    fire(c0 + 1, 1)
    drain(0)
    process(c0, 0)
    drain(1)
    process(c0 + 1, 1)

    pltpu.sync_copy(hbuf, h_out.at[pl.ds(wid * ROWS_PER_W, ROWS_PER_W)])


def _cbow_sum(words, table):
    mesh = plsc.VectorSubcoreMesh(core_axis_name="c", subcore_axis_name="s")
    kern = pl.kernel(
        _cbow_body,
        out_type=jax.ShapeDtypeStruct((B, EMB), jnp.float32),
        mesh=mesh,
        scratch_types=[
            pltpu.VMEM((R, L), jnp.int32),
            pltpu.VMEM((R, L), jnp.int32),
            pltpu.VMEM((CI, EMB), jnp.float32),
            pltpu.VMEM((CI, EMB), jnp.float32),
            pltpu.VMEM((ROWS_PER_W, EMB), jnp.float32),
            pltpu.SemaphoreType.DMA,
            pltpu.SemaphoreType.DMA,
        ],
        compiler_params=pltpu.CompilerParams(use_tc_tiling_on_sc=False),
    )
    return kern(words, table)


def _mlp_body(image_ref, h_ref, words_ref, t0_ref, w1_ref, b1_ref, out_ref):
    x = image_ref[...]
    cnt = jnp.sum((words_ref[...] == 0).astype(jnp.float32), axis=1,
                  keepdims=True)
    h = h_ref[...] - cnt * t0_ref[0:1, :]
    wa = w1_ref[:, 0:IMG_F]
    wb = w1_ref[:, IMG_F:IMG_F + EMB]
    acc = lax.dot_general(x, wa, (((1,), (1,)), ((), ())),
                          preferred_element_type=jnp.float32)
    acc = acc + lax.dot_general(h, wb, (((1,), (1,)), ((), ())),
                                preferred_element_type=jnp.float32)
    out_ref[...] = acc + b1_ref[...][None, :]


def _mlp(image, h, words, t0, W1, b1):
    BLK = 1024
    grid = (B // BLK,)
    return pl.pallas_call(
        _mlp_body,
        grid=grid,
        in_specs=[
            pl.BlockSpec((BLK, IMG_F), lambda i: (i, 0)),
            pl.BlockSpec((BLK, EMB), lambda i: (i, 0)),
            pl.BlockSpec((BLK, L), lambda i: (i, 0)),
            pl.BlockSpec((8, EMB), lambda i: (0, 0)),
            pl.BlockSpec((OUT, IMG_F + EMB), lambda i: (0, 0)),
            pl.BlockSpec((OUT,), lambda i: (0,)),
        ],
        out_specs=pl.BlockSpec((BLK, OUT), lambda i: (i, 0)),
        out_shape=jax.ShapeDtypeStruct((B, OUT), jnp.float32),
    )(image, h, words, t0, W1, b1)


@jax.jit
def kernel(words, image, table, W1, b1):
    words = words.astype(jnp.int32)
    h = _cbow_sum(words, table)
    return _mlp(image, h, words, table[0:8], W1, b1)
